# Initial kernel scaffold; baseline (speedup 1.0000x reference)
import jax
import jax.numpy as jnp
from jax.experimental import pallas as pl


def kernel(z, pos, edge_index, batch, emb, mlp_w1, mlp_b1, mlp_w2, mlp_b2, lin1_w, lin2_w, lin2_b, lin_w, lin_b, out1_w, out1_b, out2_w, out2_b):
    raise NotImplementedError("write your pallas kernel here")


# speedup vs baseline: 1.5633x; 1.5633x over previous
import functools
import math

import jax
import jax.numpy as jnp
from jax import lax
from jax.experimental import pallas as pl
from jax.experimental.pallas import tpu as pltpu
from jax.experimental.pallas import tpu_sc as plsc

N = 10000
E = 320000
H = 128
G = 50
GP = 64
NG = 16
L = 4
CUT = 10.0
VP = 128

NC = 2
NS = 16
NW = NC * NS
EPW = E // NW
WIN = 80
NWIN = EPW // WIN
SRP = 624
TAIL = N - NS * SRP

BE = 2560
GRID_E = E // BE
BN = 2000
GRID_N = N // BN

_SPACING = CUT / (G - 1)
_COEFF = -0.5 / (_SPACING * _SPACING)

f32 = jnp.float32
i32 = jnp.int32


def _ssp(x):
    return jnp.maximum(x, 0.0) + jnp.log1p(jnp.exp(-jnp.abs(x))) - math.log(2.0)


def _geom_body(px_h, py_h, pz_h, row_h, col_h, d2_h, px, py, pz, ridx, cidx, d2):
    c = lax.axis_index("c")
    s = lax.axis_index("s")
    base = (c * NS + s) * EPW
    pltpu.sync_copy(px_h, px)
    pltpu.sync_copy(py_h, py)
    pltpu.sync_copy(pz_h, pz)
    pltpu.sync_copy(row_h.at[pl.ds(base, EPW)], ridx)
    pltpu.sync_copy(col_h.at[pl.ds(base, EPW)], cidx)

    def body(i, carry):
        r = ridx[pl.ds(i * 16, 16)]
        cc = cidx[pl.ds(i * 16, 16)]
        dx = plsc.load_gather(px, [r]) - plsc.load_gather(px, [cc])
        dy = plsc.load_gather(py, [r]) - plsc.load_gather(py, [cc])
        dz = plsc.load_gather(pz, [r]) - plsc.load_gather(pz, [cc])
        d2[pl.ds(i * 16, 16)] = dx * dx + dy * dy + dz * dz
        return carry

    lax.fori_loop(0, EPW // 16, body, 0)
    pltpu.sync_copy(d2, d2_h.at[pl.ds(base, EPW)])


def _gather_body(x_h, col_h, xg_h, cidx, rows, sem):
    c = lax.axis_index("c")
    s = lax.axis_index("s")
    base = (c * NS + s) * EPW

    def body(w, carry):
        off = base + w * WIN
        pltpu.sync_copy(col_h.at[pl.ds(off, WIN)], cidx)
        pltpu.async_copy(x_h.at[cidx], rows, sem).wait()
        pltpu.sync_copy(rows, xg_h.at[pl.ds(off, WIN)])
        return carry

    lax.fori_loop(0, NWIN, body, 0)


def _scatter_body(msg_h, row_h, zero_h, agg_h, ridx, mrows, acc_sh):
    c = lax.axis_index("c")
    s = lax.axis_index("s")
    base = (c * NS + s) * EPW
    pltpu.sync_copy(zero_h.at[pl.ds(s * SRP, SRP)], acc_sh.at[pl.ds(s * SRP, SRP)])

    @pl.when(s == 0)
    def _():
        pltpu.sync_copy(zero_h.at[pl.ds(NS * SRP, TAIL)],
                        acc_sh.at[pl.ds(NS * SRP, TAIL)])

    plsc.subcore_barrier()

    def body(w, carry):
        off = base + w * WIN
        pltpu.sync_copy(row_h.at[pl.ds(off, WIN)], ridx)
        pltpu.sync_copy(msg_h.at[pl.ds(off, WIN)], mrows)
        pltpu.sync_copy(mrows, acc_sh.at[ridx], add=True)
        return carry

    lax.fori_loop(0, NWIN, body, 0)
    plsc.subcore_barrier()
    pltpu.sync_copy(acc_sh.at[pl.ds(s * SRP, SRP)],
                    agg_h.at[pl.ds(c * N + s * SRP, SRP)])

    @pl.when(s == 0)
    def _():
        pltpu.sync_copy(acc_sh.at[pl.ds(NS * SRP, TAIL)],
                        agg_h.at[pl.ds(c * N + NS * SRP, TAIL)])


@functools.lru_cache(maxsize=None)
def _sc_kernels():
    mesh = plsc.VectorSubcoreMesh(core_axis_name="c", subcore_axis_name="s",
                                  num_cores=NC, num_subcores=NS)
    geom = pl.kernel(
        _geom_body,
        out_type=jax.ShapeDtypeStruct((E,), f32),
        mesh=mesh,
        compiler_params=pltpu.CompilerParams(needs_layout_passes=False),
        scratch_types=[
            pltpu.VMEM((N,), f32),
            pltpu.VMEM((N,), f32),
            pltpu.VMEM((N,), f32),
            pltpu.VMEM((EPW,), i32),
            pltpu.VMEM((EPW,), i32),
            pltpu.VMEM((EPW,), f32),
        ],
    )
    gather = pl.kernel(
        _gather_body,
        out_type=jax.ShapeDtypeStruct((E, H), f32),
        mesh=mesh,
        scratch_types=[
            pltpu.VMEM((WIN,), i32),
            pltpu.VMEM((WIN, H), f32),
            pltpu.SemaphoreType.DMA,
        ],
    )
    scatter = pl.kernel(
        _scatter_body,
        out_type=jax.ShapeDtypeStruct((NC * N, H), f32),
        mesh=mesh,
        scratch_types=[
            pltpu.VMEM((WIN,), i32),
            pltpu.VMEM((WIN, H), f32),
            pltpu.VMEM_SHARED((N, H), f32),
        ],
    )
    return geom, gather, scatter


def _embed_kernel(z_ref, emb_ref, w1n_ref, h_ref, x_ref):
    zb = z_ref[0]
    lanes = lax.broadcasted_iota(i32, (VP, 1), 0)
    oht = (zb == lanes).astype(f32)
    h = lax.dot_general(oht, emb_ref[...], (((0,), (0,)), ((), ())),
                        preferred_element_type=f32)
    h_ref[...] = h
    x_ref[...] = jnp.dot(h, w1n_ref[...], preferred_element_type=f32)


def _filter_kernel(d2_ref, xg_ref, offs_ref, w1_ref, b1_ref, w2_ref, b2_ref,
                   msg_ref):
    d2 = d2_ref[0]
    d = jnp.sqrt(d2 + 1e-12)
    ea = jnp.exp(_COEFF * (d - offs_ref[...]) ** 2)
    t1 = _ssp(jnp.dot(ea, w1_ref[...], preferred_element_type=f32)
              + b1_ref[...])
    wf = jnp.dot(t1, w2_ref[...], preferred_element_type=f32) + b2_ref[...]
    cc = 0.5 * (jnp.cos(d * math.pi / CUT) + 1.0) * (d < CUT).astype(f32)
    msg_ref[...] = xg_ref[...] * wf * cc


def _node_kernel(a0_ref, a1_ref, h_ref, w2_ref, b2_ref, w_ref, b_ref, w1n_ref,
                 hn_ref, xn_ref):
    agg = a0_ref[0] + a1_ref[0]
    t = _ssp(jnp.dot(agg, w2_ref[...], preferred_element_type=f32) + b2_ref[...])
    x2 = jnp.dot(t, w_ref[...], preferred_element_type=f32) + b_ref[...]
    hn = h_ref[...] + x2
    hn_ref[...] = hn
    xn_ref[...] = jnp.dot(hn, w1n_ref[...], preferred_element_type=f32)


def _pool_kernel(batch_ref, h_ref, o1w_ref, o1b_ref, o2w_ref, o2b_ref,
                 out_ref, acc):
    i = pl.program_id(0)

    @pl.when(i == 0)
    def _():
        acc[...] = jnp.zeros_like(acc)

    bb = batch_ref[0]
    seg = lax.broadcasted_iota(i32, (NG, 1), 0)
    oht = (bb == seg).astype(f32)
    acc[...] += lax.dot_general(oht, h_ref[...], (((1,), (0,)), ((), ())),
                                preferred_element_type=f32)

    @pl.when(i == pl.num_programs(0) - 1)
    def _():
        t = jnp.dot(acc[...], o1w_ref[...], preferred_element_type=f32) + o1b_ref[...]
        t = t * (1.0 / (1.0 + jnp.exp(-t)))
        out_ref[...] = jnp.dot(t, o2w_ref[...], preferred_element_type=f32) + o2b_ref[...]


def _full_spec(shape):
    return pl.BlockSpec(shape, lambda *_: tuple(0 for _ in shape))


def _tc_embed(z3, embp, w1n):
    return pl.pallas_call(
        _embed_kernel,
        grid=(GRID_N,),
        in_specs=[
            pl.BlockSpec((1, 1, BN), lambda i: (i, 0, 0)),
            _full_spec((VP, H)),
            _full_spec((H, H)),
        ],
        out_specs=[
            pl.BlockSpec((BN, H), lambda i: (i, 0)),
            pl.BlockSpec((BN, H), lambda i: (i, 0)),
        ],
        out_shape=[
            jax.ShapeDtypeStruct((N, H), f32),
            jax.ShapeDtypeStruct((N, H), f32),
        ],
        compiler_params=pltpu.CompilerParams(
            dimension_semantics=("parallel",)),
    )(z3, embp, w1n)


def _tc_filter(d2c, xg, offs, w1, b1, w2, b2):
    return pl.pallas_call(
        _filter_kernel,
        grid=(GRID_E,),
        in_specs=[
            pl.BlockSpec((1, BE, 1), lambda i: (i, 0, 0)),
            pl.BlockSpec((BE, H), lambda i: (i, 0)),
            _full_spec((1, GP)),
            _full_spec((GP, H)),
            _full_spec((1, H)),
            _full_spec((H, H)),
            _full_spec((1, H)),
        ],
        out_specs=pl.BlockSpec((BE, H), lambda i: (i, 0)),
        out_shape=jax.ShapeDtypeStruct((E, H), f32),
        compiler_params=pltpu.CompilerParams(
            dimension_semantics=("parallel",)),
    )(d2c, xg, offs, w1, b1, w2, b2)


def _tc_node(aggp, h, w2, b2, w, b, w1n):
    aggp = (aggp, aggp)
    return pl.pallas_call(
        _node_kernel,
        grid=(GRID_N,),
        in_specs=[
            pl.BlockSpec((1, BN, H), lambda i: (0, i, 0)),
            pl.BlockSpec((1, BN, H), lambda i: (1, i, 0)),
            pl.BlockSpec((BN, H), lambda i: (i, 0)),
            _full_spec((H, H)),
            _full_spec((1, H)),
            _full_spec((H, H)),
            _full_spec((1, H)),
            _full_spec((H, H)),
        ],
        out_specs=[
            pl.BlockSpec((BN, H), lambda i: (i, 0)),
            pl.BlockSpec((BN, H), lambda i: (i, 0)),
        ],
        out_shape=[
            jax.ShapeDtypeStruct((N, H), f32),
            jax.ShapeDtypeStruct((N, H), f32),
        ],
        compiler_params=pltpu.CompilerParams(
            dimension_semantics=("parallel",)),
    )(aggp[0], aggp[1], h, w2, b2, w, b, w1n)


def _tc_pool(batch3, h, o1w, o1b, o2w, o2b):
    return pl.pallas_call(
        _pool_kernel,
        grid=(GRID_N,),
        in_specs=[
            pl.BlockSpec((1, 1, BN), lambda i: (i, 0, 0)),
            pl.BlockSpec((BN, H), lambda i: (i, 0)),
            _full_spec((H, H)),
            _full_spec((1, H)),
            _full_spec((H, H)),
            _full_spec((1, H)),
        ],
        out_specs=pl.BlockSpec((NG, H), lambda i: (0, 0)),
        out_shape=jax.ShapeDtypeStruct((NG, H), f32),
        scratch_shapes=[pltpu.VMEM((NG, H), f32)],
        compiler_params=pltpu.CompilerParams(
            dimension_semantics=("arbitrary",)),
    )(batch3, h, o1w, o1b, o2w, o2b)


def kernel(z, pos, edge_index, batch, emb, mlp_w1, mlp_b1, mlp_w2, mlp_b2,
           lin1_w, lin2_w, lin2_b, lin_w, lin_b, out1_w, out1_b, out2_w,
           out2_b):
    row = edge_index[0].astype(i32)
    col = edge_index[1].astype(i32)
    px = pos[:, 0]
    py = pos[:, 1]
    pz = pos[:, 2]
    z3 = z.astype(i32).reshape(GRID_N, 1, BN)
    batch3 = batch.astype(i32).reshape(GRID_N, 1, BN)

    embp = jnp.pad(emb, ((0, VP - emb.shape[0]), (0, 0)))
    w1p = jnp.pad(mlp_w1, ((0, 0), (0, GP - G), (0, 0)))
    offs = jnp.linspace(0.0, CUT, G, dtype=f32)
    offsp = jnp.pad(offs, (0, GP - G), constant_values=1e6).reshape(1, GP)
    o1wp = jnp.pad(out1_w, ((0, 0), (0, H - out1_w.shape[1])))
    o1bp = jnp.pad(out1_b, (0, H - out1_b.shape[0])).reshape(1, H)
    o2wp = jnp.pad(out2_w, ((0, H - out2_w.shape[0]), (0, H - out2_w.shape[1])))
    o2bp = jnp.pad(out2_b, (0, H - out2_b.shape[0])).reshape(1, H)
    zeros_nh = jnp.zeros((N, H), f32)
    _geom, _gather_rows, _scatter_add = _sc_kernels()

    d2 = _geom(px, py, pz, row, col)
    d2c = d2.reshape(GRID_E, BE, 1)

    h, x = _tc_embed(z3, embp, lin1_w[0])

    for l in range(L):
        xg = _gather_rows(x, col)
        msg = _tc_filter(d2c, xg, offsp, w1p[l], mlp_b1[l].reshape(1, H),
                         mlp_w2[l], mlp_b2[l].reshape(1, H))
        aggp = _scatter_add(msg, row, zeros_nh)
        w1n = lin1_w[(l + 1) % L]
        h, x = _tc_node(aggp.reshape(NC, N, H), h, lin2_w[l],
                        lin2_b[l].reshape(1, H), lin_w[l],
                        lin_b[l].reshape(1, H), w1n)

    outp = _tc_pool(batch3, h, o1wp, o1bp, o2wp, o2bp)
    return outp[:, :1]


# speedup vs baseline: 2.2300x; 1.4265x over previous
import functools
import math

import jax
import jax.numpy as jnp
from jax import lax
from jax.experimental import pallas as pl
from jax.experimental.pallas import tpu as pltpu
from jax.experimental.pallas import tpu_sc as plsc

N = 10000
E = 320000
H = 128
G = 50
GP = 64
NG = 16
L = 4
CUT = 10.0
VP = 128

NC = 2
NS = 16
NW = NC * NS
EPW = E // NW
WIN = 80
NWIN = EPW // WIN
SRP = 624
TAIL = N - NS * SRP

BE = 2560
GRID_E = E // BE
BN = 2000
GRID_N = N // BN

_SPACING = CUT / (G - 1)
_COEFF = -0.5 / (_SPACING * _SPACING)

f32 = jnp.float32
i32 = jnp.int32


def _ssp(x):
    return jnp.maximum(x, 0.0) + jnp.log1p(jnp.exp(-jnp.abs(x))) - math.log(2.0)


def _geom_body(px_h, py_h, pz_h, row_h, col_h, d2_h, px, py, pz, ridx, cidx, d2):
    c = lax.axis_index("c")
    s = lax.axis_index("s")
    base = (c * NS + s) * EPW
    pltpu.sync_copy(px_h, px)
    pltpu.sync_copy(py_h, py)
    pltpu.sync_copy(pz_h, pz)
    pltpu.sync_copy(row_h.at[pl.ds(base, EPW)], ridx)
    pltpu.sync_copy(col_h.at[pl.ds(base, EPW)], cidx)

    def body(i, carry):
        r = ridx[pl.ds(i * 16, 16)]
        cc = cidx[pl.ds(i * 16, 16)]
        dx = plsc.load_gather(px, [r]) - plsc.load_gather(px, [cc])
        dy = plsc.load_gather(py, [r]) - plsc.load_gather(py, [cc])
        dz = plsc.load_gather(pz, [r]) - plsc.load_gather(pz, [cc])
        d2[pl.ds(i * 16, 16)] = dx * dx + dy * dy + dz * dz
        return carry

    lax.fori_loop(0, EPW // 16, body, 0)
    pltpu.sync_copy(d2, d2_h.at[pl.ds(base, EPW)])


def _gather_body(x_h, col_h, xg_h, cidx, rows, sem):
    c = lax.axis_index("c")
    s = lax.axis_index("s")
    base = (c * NS + s) * EPW

    def body(w, carry):
        off = base + w * WIN
        pltpu.sync_copy(col_h.at[pl.ds(off, WIN)], cidx)
        pltpu.async_copy(x_h.at[cidx], rows, sem).wait()
        pltpu.sync_copy(rows, xg_h.at[pl.ds(off, WIN)])
        return carry

    lax.fori_loop(0, NWIN, body, 0)


def _scatter_body(msg_h, row_h, zero_h, agg_h, ridx, mrows, acc_sh):
    c = lax.axis_index("c")
    s = lax.axis_index("s")
    base = (c * NS + s) * EPW
    pltpu.sync_copy(zero_h.at[pl.ds(s * SRP, SRP)], acc_sh.at[pl.ds(s * SRP, SRP)])

    @pl.when(s == 0)
    def _():
        pltpu.sync_copy(zero_h.at[pl.ds(NS * SRP, TAIL)],
                        acc_sh.at[pl.ds(NS * SRP, TAIL)])

    plsc.subcore_barrier()

    def body(w, carry):
        off = base + w * WIN
        pltpu.sync_copy(row_h.at[pl.ds(off, WIN)], ridx)
        pltpu.sync_copy(msg_h.at[pl.ds(off, WIN)], mrows)
        pltpu.sync_copy(mrows, acc_sh.at[ridx], add=True)
        return carry

    lax.fori_loop(0, NWIN, body, 0)
    plsc.subcore_barrier()
    pltpu.sync_copy(acc_sh.at[pl.ds(s * SRP, SRP)],
                    agg_h.at[pl.ds(c * N + s * SRP, SRP)])

    @pl.when(s == 0)
    def _():
        pltpu.sync_copy(acc_sh.at[pl.ds(NS * SRP, TAIL)],
                        agg_h.at[pl.ds(c * N + NS * SRP, TAIL)])


@functools.lru_cache(maxsize=None)
def _sc_kernels():
    mesh = plsc.VectorSubcoreMesh(core_axis_name="c", subcore_axis_name="s",
                                  num_cores=NC, num_subcores=NS)
    geom = pl.kernel(
        _geom_body,
        out_type=jax.ShapeDtypeStruct((E,), f32),
        mesh=mesh,
        compiler_params=pltpu.CompilerParams(needs_layout_passes=False),
        scratch_types=[
            pltpu.VMEM((N,), f32),
            pltpu.VMEM((N,), f32),
            pltpu.VMEM((N,), f32),
            pltpu.VMEM((EPW,), i32),
            pltpu.VMEM((EPW,), i32),
            pltpu.VMEM((EPW,), f32),
        ],
    )
    gather = pl.kernel(
        _gather_body,
        out_type=jax.ShapeDtypeStruct((E, H), f32),
        mesh=mesh,
        scratch_types=[
            pltpu.VMEM((WIN,), i32),
            pltpu.VMEM((WIN, H), f32),
            pltpu.SemaphoreType.DMA,
        ],
    )
    scatter = pl.kernel(
        _scatter_body,
        out_type=jax.ShapeDtypeStruct((NC * N, H), f32),
        mesh=mesh,
        scratch_types=[
            pltpu.VMEM((WIN,), i32),
            pltpu.VMEM((WIN, H), f32),
            pltpu.VMEM_SHARED((N, H), f32),
        ],
    )
    return geom, gather, scatter


def _embed_kernel(z_ref, emb_ref, w1n_ref, h_ref, x_ref):
    zb = z_ref[0]
    lanes = lax.broadcasted_iota(i32, (VP, 1), 0)
    oht = (zb == lanes).astype(f32)
    h = lax.dot_general(oht, emb_ref[...], (((0,), (0,)), ((), ())),
                        preferred_element_type=f32)
    h_ref[...] = h
    x_ref[...] = jnp.dot(h, w1n_ref[...], preferred_element_type=f32)


def _ea_kernel(d2_ref, offs_ref, ea_ref, cc_ref):
    d2 = d2_ref[0]
    d = jnp.sqrt(d2 + 1e-12)
    ea_ref[...] = jnp.exp(_COEFF * (d - offs_ref[...]) ** 2)
    cc_ref[...] = 0.5 * (jnp.cos(d * math.pi / CUT) + 1.0) * (d < CUT).astype(f32)


def _filter_kernel(ea_ref, cc_ref, xg_ref, w1_ref, b1_ref, w2_ref, b2_ref,
                   msg_ref):
    t1 = _ssp(jnp.dot(ea_ref[...], w1_ref[...], preferred_element_type=f32)
              + b1_ref[...])
    wf = jnp.dot(t1, w2_ref[...], preferred_element_type=f32) + b2_ref[...]
    msg_ref[...] = xg_ref[...] * wf * cc_ref[...]


def _node_kernel(a0_ref, a1_ref, h_ref, w2_ref, b2_ref, w_ref, b_ref, w1n_ref,
                 hn_ref, xn_ref):
    agg = a0_ref[0] + a1_ref[0]
    t = _ssp(jnp.dot(agg, w2_ref[...], preferred_element_type=f32) + b2_ref[...])
    x2 = jnp.dot(t, w_ref[...], preferred_element_type=f32) + b_ref[...]
    hn = h_ref[...] + x2
    hn_ref[...] = hn
    xn_ref[...] = jnp.dot(hn, w1n_ref[...], preferred_element_type=f32)


def _pool_kernel(batch_ref, h_ref, o1w_ref, o1b_ref, o2w_ref, o2b_ref,
                 out_ref, acc):
    i = pl.program_id(0)

    @pl.when(i == 0)
    def _():
        acc[...] = jnp.zeros_like(acc)

    bb = batch_ref[0]
    seg = lax.broadcasted_iota(i32, (NG, 1), 0)
    oht = (bb == seg).astype(f32)
    acc[...] += lax.dot_general(oht, h_ref[...], (((1,), (0,)), ((), ())),
                                preferred_element_type=f32)

    @pl.when(i == pl.num_programs(0) - 1)
    def _():
        t = jnp.dot(acc[...], o1w_ref[...], preferred_element_type=f32) + o1b_ref[...]
        t = t * (1.0 / (1.0 + jnp.exp(-t)))
        out_ref[...] = jnp.dot(t, o2w_ref[...], preferred_element_type=f32) + o2b_ref[...]


def _full_spec(shape):
    return pl.BlockSpec(shape, lambda *_: tuple(0 for _ in shape))


def _tc_embed(z3, embp, w1n):
    return pl.pallas_call(
        _embed_kernel,
        grid=(GRID_N,),
        in_specs=[
            pl.BlockSpec((1, 1, BN), lambda i: (i, 0, 0)),
            _full_spec((VP, H)),
            _full_spec((H, H)),
        ],
        out_specs=[
            pl.BlockSpec((BN, H), lambda i: (i, 0)),
            pl.BlockSpec((BN, H), lambda i: (i, 0)),
        ],
        out_shape=[
            jax.ShapeDtypeStruct((N, H), f32),
            jax.ShapeDtypeStruct((N, H), f32),
        ],
        compiler_params=pltpu.CompilerParams(
            dimension_semantics=("parallel",)),
    )(z3, embp, w1n)


def _tc_ea(d2c, offs):
    return pl.pallas_call(
        _ea_kernel,
        grid=(GRID_E,),
        in_specs=[
            pl.BlockSpec((1, BE, 1), lambda i: (i, 0, 0)),
            _full_spec((1, GP)),
        ],
        out_specs=[
            pl.BlockSpec((BE, GP), lambda i: (i, 0)),
            pl.BlockSpec((BE, 1), lambda i: (i, 0)),
        ],
        out_shape=[
            jax.ShapeDtypeStruct((E, GP), f32),
            jax.ShapeDtypeStruct((E, 1), f32),
        ],
        compiler_params=pltpu.CompilerParams(
            dimension_semantics=("parallel",)),
    )(d2c, offs)


def _tc_filter(ea, cc, xg, w1, b1, w2, b2):
    return pl.pallas_call(
        _filter_kernel,
        grid=(GRID_E,),
        in_specs=[
            pl.BlockSpec((BE, GP), lambda i: (i, 0)),
            pl.BlockSpec((BE, 1), lambda i: (i, 0)),
            pl.BlockSpec((BE, H), lambda i: (i, 0)),
            _full_spec((GP, H)),
            _full_spec((1, H)),
            _full_spec((H, H)),
            _full_spec((1, H)),
        ],
        out_specs=pl.BlockSpec((BE, H), lambda i: (i, 0)),
        out_shape=jax.ShapeDtypeStruct((E, H), f32),
        compiler_params=pltpu.CompilerParams(
            dimension_semantics=("parallel",)),
    )(ea, cc, xg, w1, b1, w2, b2)


def _tc_node(aggp, h, w2, b2, w, b, w1n):
    aggp = (aggp, aggp)
    return pl.pallas_call(
        _node_kernel,
        grid=(GRID_N,),
        in_specs=[
            pl.BlockSpec((1, BN, H), lambda i: (0, i, 0)),
            pl.BlockSpec((1, BN, H), lambda i: (1, i, 0)),
            pl.BlockSpec((BN, H), lambda i: (i, 0)),
            _full_spec((H, H)),
            _full_spec((1, H)),
            _full_spec((H, H)),
            _full_spec((1, H)),
            _full_spec((H, H)),
        ],
        out_specs=[
            pl.BlockSpec((BN, H), lambda i: (i, 0)),
            pl.BlockSpec((BN, H), lambda i: (i, 0)),
        ],
        out_shape=[
            jax.ShapeDtypeStruct((N, H), f32),
            jax.ShapeDtypeStruct((N, H), f32),
        ],
        compiler_params=pltpu.CompilerParams(
            dimension_semantics=("parallel",)),
    )(aggp[0], aggp[1], h, w2, b2, w, b, w1n)


def _tc_pool(batch3, h, o1w, o1b, o2w, o2b):
    return pl.pallas_call(
        _pool_kernel,
        grid=(GRID_N,),
        in_specs=[
            pl.BlockSpec((1, 1, BN), lambda i: (i, 0, 0)),
            pl.BlockSpec((BN, H), lambda i: (i, 0)),
            _full_spec((H, H)),
            _full_spec((1, H)),
            _full_spec((H, H)),
            _full_spec((1, H)),
        ],
        out_specs=pl.BlockSpec((NG, H), lambda i: (0, 0)),
        out_shape=jax.ShapeDtypeStruct((NG, H), f32),
        scratch_shapes=[pltpu.VMEM((NG, H), f32)],
        compiler_params=pltpu.CompilerParams(
            dimension_semantics=("arbitrary",)),
    )(batch3, h, o1w, o1b, o2w, o2b)


def kernel(z, pos, edge_index, batch, emb, mlp_w1, mlp_b1, mlp_w2, mlp_b2,
           lin1_w, lin2_w, lin2_b, lin_w, lin_b, out1_w, out1_b, out2_w,
           out2_b):
    row = edge_index[0].astype(i32)
    col = edge_index[1].astype(i32)
    px = pos[:, 0]
    py = pos[:, 1]
    pz = pos[:, 2]
    z3 = z.astype(i32).reshape(GRID_N, 1, BN)
    batch3 = batch.astype(i32).reshape(GRID_N, 1, BN)

    embp = jnp.pad(emb, ((0, VP - emb.shape[0]), (0, 0)))
    w1p = jnp.pad(mlp_w1, ((0, 0), (0, GP - G), (0, 0)))
    offs = jnp.linspace(0.0, CUT, G, dtype=f32)
    offsp = jnp.pad(offs, (0, GP - G), constant_values=1e6).reshape(1, GP)
    o1wp = jnp.pad(out1_w, ((0, 0), (0, H - out1_w.shape[1])))
    o1bp = jnp.pad(out1_b, (0, H - out1_b.shape[0])).reshape(1, H)
    o2wp = jnp.pad(out2_w, ((0, H - out2_w.shape[0]), (0, H - out2_w.shape[1])))
    o2bp = jnp.pad(out2_b, (0, H - out2_b.shape[0])).reshape(1, H)
    zeros_nh = jnp.zeros((N, H), f32)
    _geom, _gather_rows, _scatter_add = _sc_kernels()

    d2 = _geom(px, py, pz, row, col)
    d2c = d2.reshape(GRID_E, BE, 1)

    h, x = _tc_embed(z3, embp, lin1_w[0])

    ea, cc = _tc_ea(d2c, offsp)

    for l in range(L):
        xg = _gather_rows(x, col)
        msg = _tc_filter(ea, cc, xg, w1p[l], mlp_b1[l].reshape(1, H),
                         mlp_w2[l], mlp_b2[l].reshape(1, H))
        aggp = _scatter_add(msg, row, zeros_nh)
        w1n = lin1_w[(l + 1) % L]
        h, x = _tc_node(aggp.reshape(NC, N, H), h, lin2_w[l],
                        lin2_b[l].reshape(1, H), lin_w[l],
                        lin_b[l].reshape(1, H), w1n)

    outp = _tc_pool(batch3, h, o1wp, o1bp, o2wp, o2bp)
    return outp[:, :1]


# speedup vs baseline: 2.4951x; 1.1188x over previous
import functools
import math

import jax
import jax.numpy as jnp
from jax import lax
from jax.experimental import pallas as pl
from jax.experimental.pallas import tpu as pltpu
from jax.experimental.pallas import tpu_sc as plsc

N = 10000
E = 320000
H = 128
G = 50
GP = 64
NG = 16
L = 4
CUT = 10.0
VP = 128

NC = 2
NS = 16
NW = NC * NS
EPW = E // NW
WIN = 80
NWIN = EPW // WIN
SRP = 624
TAIL = N - NS * SRP

EH1 = 192000
EH2 = E - EH1
EHS = (EH1, EH2)
HOFFS = (0, EH1)

BE = 2560
GRID_E = E // BE
BN = 2000
GRID_N = N // BN

_SPACING = CUT / (G - 1)
_COEFF = -0.5 / (_SPACING * _SPACING)

f32 = jnp.float32
i32 = jnp.int32


def _dotd(a, b):
    return jnp.dot(a.astype(jnp.bfloat16), b.astype(jnp.bfloat16),
                   preferred_element_type=f32)


def _ssp(x):
    return jnp.maximum(x, 0.0) + jnp.log1p(jnp.exp(-jnp.abs(x))) - math.log(2.0)


def _geom_body(px_h, py_h, pz_h, row_h, col_h, d2_h, px, py, pz, ridx, cidx, d2):
    c = lax.axis_index("c")
    s = lax.axis_index("s")
    base = (c * NS + s) * EPW
    pltpu.sync_copy(px_h, px)
    pltpu.sync_copy(py_h, py)
    pltpu.sync_copy(pz_h, pz)
    pltpu.sync_copy(row_h.at[pl.ds(base, EPW)], ridx)
    pltpu.sync_copy(col_h.at[pl.ds(base, EPW)], cidx)

    def body(i, carry):
        r = ridx[pl.ds(i * 16, 16)]
        cc = cidx[pl.ds(i * 16, 16)]
        dx = plsc.load_gather(px, [r]) - plsc.load_gather(px, [cc])
        dy = plsc.load_gather(py, [r]) - plsc.load_gather(py, [cc])
        dz = plsc.load_gather(pz, [r]) - plsc.load_gather(pz, [cc])
        d2[pl.ds(i * 16, 16)] = dx * dx + dy * dy + dz * dz
        return carry

    lax.fori_loop(0, EPW // 16, body, 0)
    pltpu.sync_copy(d2, d2_h.at[pl.ds(base, EPW)])


def _gather_body(x_h, col_h, xg_h, cidx, rows, sem, *, hoff, epw):
    c = lax.axis_index("c")
    s = lax.axis_index("s")
    base = (c * NS + s) * epw

    def body(w, carry):
        off = base + w * WIN
        pltpu.sync_copy(col_h.at[pl.ds(hoff + off, WIN)], cidx)
        pltpu.async_copy(x_h.at[cidx], rows, sem).wait()
        pltpu.sync_copy(rows, xg_h.at[pl.ds(off, WIN)])
        return carry

    lax.fori_loop(0, epw // WIN, body, 0)


def _scatter_body(msg_h, row_h, zero_h, agg_h, ridx, mrows, acc_sh, *, hoff,
                  epw):
    c = lax.axis_index("c")
    s = lax.axis_index("s")
    base = (c * NS + s) * epw
    pltpu.sync_copy(zero_h.at[pl.ds(s * SRP, SRP)], acc_sh.at[pl.ds(s * SRP, SRP)])

    @pl.when(s == 0)
    def _():
        pltpu.sync_copy(zero_h.at[pl.ds(NS * SRP, TAIL)],
                        acc_sh.at[pl.ds(NS * SRP, TAIL)])

    plsc.subcore_barrier()

    def body(w, carry):
        off = base + w * WIN
        pltpu.sync_copy(row_h.at[pl.ds(hoff + off, WIN)], ridx)
        pltpu.sync_copy(msg_h.at[pl.ds(off, WIN)], mrows)
        pltpu.sync_copy(mrows, acc_sh.at[ridx], add=True)
        return carry

    lax.fori_loop(0, epw // WIN, body, 0)
    plsc.subcore_barrier()
    pltpu.sync_copy(acc_sh.at[pl.ds(s * SRP, SRP)],
                    agg_h.at[pl.ds(c * N + s * SRP, SRP)])

    @pl.when(s == 0)
    def _():
        pltpu.sync_copy(acc_sh.at[pl.ds(NS * SRP, TAIL)],
                        agg_h.at[pl.ds(c * N + NS * SRP, TAIL)])


@functools.lru_cache(maxsize=None)
def _sc_kernels():
    mesh = plsc.VectorSubcoreMesh(core_axis_name="c", subcore_axis_name="s",
                                  num_cores=NC, num_subcores=NS)
    geom = pl.kernel(
        _geom_body,
        out_type=jax.ShapeDtypeStruct((E,), f32),
        mesh=mesh,
        compiler_params=pltpu.CompilerParams(needs_layout_passes=False),
        scratch_types=[
            pltpu.VMEM((N,), f32),
            pltpu.VMEM((N,), f32),
            pltpu.VMEM((N,), f32),
            pltpu.VMEM((EPW,), i32),
            pltpu.VMEM((EPW,), i32),
            pltpu.VMEM((EPW,), f32),
        ],
    )
    gathers = []
    scatters = []
    for hoff, eh in zip(HOFFS, EHS):
        epw = eh // NW
        gathers.append(pl.kernel(
            functools.partial(_gather_body, hoff=hoff, epw=epw),
            out_type=jax.ShapeDtypeStruct((eh, H), f32),
            mesh=mesh,
            scratch_types=[
                pltpu.VMEM((WIN,), i32),
                pltpu.VMEM((WIN, H), f32),
                pltpu.SemaphoreType.DMA,
            ],
        ))
        scatters.append(pl.kernel(
            functools.partial(_scatter_body, hoff=hoff, epw=epw),
            out_type=jax.ShapeDtypeStruct((NC * N, H), f32),
            mesh=mesh,
            scratch_types=[
                pltpu.VMEM((WIN,), i32),
                pltpu.VMEM((WIN, H), f32),
                pltpu.VMEM_SHARED((N, H), f32),
            ],
        ))
    return geom, gathers, scatters


def _embed_kernel(z_ref, emb_ref, w1n_ref, h_ref, x_ref):
    zb = z_ref[0]
    lanes = lax.broadcasted_iota(i32, (VP, 1), 0)
    oht = (zb == lanes).astype(f32)
    h = lax.dot_general(oht, emb_ref[...], (((0,), (0,)), ((), ())),
                        preferred_element_type=f32,
                        precision=lax.Precision.HIGHEST)
    h_ref[...] = h
    x_ref[...] = _dotd(h, w1n_ref[...])


def _ea_kernel(d2_ref, offs_ref, ea_ref, cc_ref):
    d2 = d2_ref[0]
    d = jnp.sqrt(d2 + 1e-12)
    ea_ref[...] = jnp.exp(_COEFF * (d - offs_ref[...]) ** 2)
    cc_ref[...] = 0.5 * (jnp.cos(d * math.pi / CUT) + 1.0) * (d < CUT).astype(f32)


def _filter_kernel(ea_ref, cc_ref, xg_ref, w1_ref, b1_ref, w2_ref, b2_ref,
                   msg_ref):
    t1 = _ssp(_dotd(ea_ref[...], w1_ref[...]) + b1_ref[...])
    wf = _dotd(t1, w2_ref[...]) + b2_ref[...]
    msg_ref[...] = xg_ref[...] * wf * cc_ref[...]


def _node_kernel(a0_ref, a1_ref, a2_ref, a3_ref, h_ref, w2_ref, b2_ref,
                 w_ref, b_ref, w1n_ref, hn_ref, xn_ref):
    agg = (a0_ref[0] + a1_ref[0]) + (a2_ref[0] + a3_ref[0])
    t = _ssp(_dotd(agg, w2_ref[...]) + b2_ref[...])
    x2 = _dotd(t, w_ref[...]) + b_ref[...]
    hn = h_ref[...] + x2
    hn_ref[...] = hn
    xn_ref[...] = _dotd(hn, w1n_ref[...])


def _pool_kernel(batch_ref, h_ref, o1w_ref, o1b_ref, o2w_ref, o2b_ref,
                 out_ref, acc):
    i = pl.program_id(0)

    @pl.when(i == 0)
    def _():
        acc[...] = jnp.zeros_like(acc)

    bb = batch_ref[0]
    seg = lax.broadcasted_iota(i32, (NG, 1), 0)
    oht = (bb == seg).astype(f32)
    acc[...] += lax.dot_general(oht, h_ref[...], (((1,), (0,)), ((), ())),
                                preferred_element_type=f32, precision=lax.Precision.HIGHEST)

    @pl.when(i == pl.num_programs(0) - 1)
    def _():
        t = _dotd(acc[...], o1w_ref[...]) + o1b_ref[...]
        t = t * (1.0 / (1.0 + jnp.exp(-t)))
        out_ref[...] = _dotd(t, o2w_ref[...]) + o2b_ref[...]


def _full_spec(shape):
    return pl.BlockSpec(shape, lambda *_: tuple(0 for _ in shape))


def _tc_embed(z3, embp, w1n):
    return pl.pallas_call(
        _embed_kernel,
        grid=(GRID_N,),
        in_specs=[
            pl.BlockSpec((1, 1, BN), lambda i: (i, 0, 0)),
            _full_spec((VP, H)),
            _full_spec((H, H)),
        ],
        out_specs=[
            pl.BlockSpec((BN, H), lambda i: (i, 0)),
            pl.BlockSpec((BN, H), lambda i: (i, 0)),
        ],
        out_shape=[
            jax.ShapeDtypeStruct((N, H), f32),
            jax.ShapeDtypeStruct((N, H), f32),
        ],
        compiler_params=pltpu.CompilerParams(
            dimension_semantics=("parallel",)),
    )(z3, embp, w1n)


def _tc_ea(d2c, offs):
    return pl.pallas_call(
        _ea_kernel,
        grid=(GRID_E,),
        in_specs=[
            pl.BlockSpec((1, BE, 1), lambda i: (i, 0, 0)),
            _full_spec((1, GP)),
        ],
        out_specs=[
            pl.BlockSpec((BE, GP), lambda i: (i, 0)),
            pl.BlockSpec((BE, 1), lambda i: (i, 0)),
        ],
        out_shape=[
            jax.ShapeDtypeStruct((E, GP), f32),
            jax.ShapeDtypeStruct((E, 1), f32),
        ],
        compiler_params=pltpu.CompilerParams(
            dimension_semantics=("parallel",)),
    )(d2c, offs)


def _tc_filter(ea, cc, xg, w1, b1, w2, b2, half):
    hblk = HOFFS[half] // BE
    eh = EHS[half]
    return pl.pallas_call(
        _filter_kernel,
        grid=(eh // BE,),
        in_specs=[
            pl.BlockSpec((BE, GP), lambda i: (i + hblk, 0)),
            pl.BlockSpec((BE, 1), lambda i: (i + hblk, 0)),
            pl.BlockSpec((BE, H), lambda i: (i, 0)),
            _full_spec((GP, H)),
            _full_spec((1, H)),
            _full_spec((H, H)),
            _full_spec((1, H)),
        ],
        out_specs=pl.BlockSpec((BE, H), lambda i: (i, 0)),
        out_shape=jax.ShapeDtypeStruct((eh, H), f32),
        compiler_params=pltpu.CompilerParams(
            dimension_semantics=("parallel",)),
    )(ea, cc, xg, w1, b1, w2, b2)


def _tc_node(agg1, agg2, h, w2, b2, w, b, w1n):
    return pl.pallas_call(
        _node_kernel,
        grid=(GRID_N,),
        in_specs=[
            pl.BlockSpec((1, BN, H), lambda i: (0, i, 0)),
            pl.BlockSpec((1, BN, H), lambda i: (1, i, 0)),
            pl.BlockSpec((1, BN, H), lambda i: (0, i, 0)),
            pl.BlockSpec((1, BN, H), lambda i: (1, i, 0)),
            pl.BlockSpec((BN, H), lambda i: (i, 0)),
            _full_spec((H, H)),
            _full_spec((1, H)),
            _full_spec((H, H)),
            _full_spec((1, H)),
            _full_spec((H, H)),
        ],
        out_specs=[
            pl.BlockSpec((BN, H), lambda i: (i, 0)),
            pl.BlockSpec((BN, H), lambda i: (i, 0)),
        ],
        out_shape=[
            jax.ShapeDtypeStruct((N, H), f32),
            jax.ShapeDtypeStruct((N, H), f32),
        ],
        compiler_params=pltpu.CompilerParams(
            dimension_semantics=("parallel",)),
    )(agg1, agg1, agg2, agg2, h, w2, b2, w, b, w1n)


def _tc_pool(batch3, h, o1w, o1b, o2w, o2b):
    return pl.pallas_call(
        _pool_kernel,
        grid=(GRID_N,),
        in_specs=[
            pl.BlockSpec((1, 1, BN), lambda i: (i, 0, 0)),
            pl.BlockSpec((BN, H), lambda i: (i, 0)),
            _full_spec((H, H)),
            _full_spec((1, H)),
            _full_spec((H, H)),
            _full_spec((1, H)),
        ],
        out_specs=pl.BlockSpec((NG, H), lambda i: (0, 0)),
        out_shape=jax.ShapeDtypeStruct((NG, H), f32),
        scratch_shapes=[pltpu.VMEM((NG, H), f32)],
        compiler_params=pltpu.CompilerParams(
            dimension_semantics=("arbitrary",)),
    )(batch3, h, o1w, o1b, o2w, o2b)


def kernel(z, pos, edge_index, batch, emb, mlp_w1, mlp_b1, mlp_w2, mlp_b2,
           lin1_w, lin2_w, lin2_b, lin_w, lin_b, out1_w, out1_b, out2_w,
           out2_b):
    row = edge_index[0].astype(i32)
    col = edge_index[1].astype(i32)
    px = pos[:, 0]
    py = pos[:, 1]
    pz = pos[:, 2]
    z3 = z.astype(i32).reshape(GRID_N, 1, BN)
    batch3 = batch.astype(i32).reshape(GRID_N, 1, BN)

    embp = jnp.pad(emb, ((0, VP - emb.shape[0]), (0, 0)))
    w1p = jnp.pad(mlp_w1, ((0, 0), (0, GP - G), (0, 0)))
    offs = jnp.linspace(0.0, CUT, G, dtype=f32)
    offsp = jnp.pad(offs, (0, GP - G), constant_values=1e6).reshape(1, GP)
    o1wp = jnp.pad(out1_w, ((0, 0), (0, H - out1_w.shape[1])))
    o1bp = jnp.pad(out1_b, (0, H - out1_b.shape[0])).reshape(1, H)
    o2wp = jnp.pad(out2_w, ((0, H - out2_w.shape[0]), (0, H - out2_w.shape[1])))
    o2bp = jnp.pad(out2_b, (0, H - out2_b.shape[0])).reshape(1, H)
    zeros_nh = jnp.zeros((N, H), f32)
    _geom, _gather_rows, _scatter_add = _sc_kernels()

    d2 = _geom(px, py, pz, row, col)
    d2c = d2.reshape(GRID_E, BE, 1)

    h, x = _tc_embed(z3, embp, lin1_w[0])

    ea, cc = _tc_ea(d2c, offsp)

    for l in range(L):
        b1l = mlp_b1[l].reshape(1, H)
        b2l = mlp_b2[l].reshape(1, H)
        xg1 = _gather_rows[0](x, col)
        msg1 = _tc_filter(ea, cc, xg1, w1p[l], b1l, mlp_w2[l], b2l, 0)
        xg2 = _gather_rows[1](x, col)
        agg1 = _scatter_add[0](msg1, row, zeros_nh)
        msg2 = _tc_filter(ea, cc, xg2, w1p[l], b1l, mlp_w2[l], b2l, 1)
        agg2 = _scatter_add[1](msg2, row, zeros_nh)
        w1n = lin1_w[(l + 1) % L]
        h, x = _tc_node(agg1.reshape(NC, N, H), agg2.reshape(NC, N, H), h,
                        lin2_w[l], lin2_b[l].reshape(1, H), lin_w[l],
                        lin_b[l].reshape(1, H), w1n)

    outp = _tc_pool(batch3, h, o1wp, o1bp, o2wp, o2bp)
    return outp[:, :1]


# speedup vs baseline: 3.2304x; 1.2947x over previous
import functools
import math

import jax
import jax.numpy as jnp
from jax import lax
from jax.experimental import pallas as pl
from jax.experimental.pallas import tpu as pltpu
from jax.experimental.pallas import tpu_sc as plsc

N = 10000
E = 320000
H = 128
G = 50
GP = 64
NG = 16
L = 4
CUT = 10.0
VP = 128

NC = 2
NS = 16
NW = NC * NS
EPW = E // NW
WIN = 80
NWIN = EPW // WIN
SRP = 624
TAIL = N - NS * SRP

EH1 = 192000
EH2 = E - EH1
EHS = (EH1, EH2)
HOFFS = (0, EH1)

BE = 2560
GRID_E = E // BE
BN = 2000
GRID_N = N // BN

_SPACING = CUT / (G - 1)
_COEFF = -0.5 / (_SPACING * _SPACING)

f32 = jnp.float32
i32 = jnp.int32


def _dotd(a, b):
    return jnp.dot(a.astype(jnp.bfloat16), b.astype(jnp.bfloat16),
                   preferred_element_type=f32)


def _ssp(x):
    return jnp.maximum(x, 0.0) + jnp.log1p(jnp.exp(-jnp.abs(x))) - math.log(2.0)


def _geom_body(px_h, py_h, pz_h, row_h, col_h, d2_h, px, py, pz, ridx, cidx, d2):
    c = lax.axis_index("c")
    s = lax.axis_index("s")
    base = (c * NS + s) * EPW
    pltpu.sync_copy(px_h, px)
    pltpu.sync_copy(py_h, py)
    pltpu.sync_copy(pz_h, pz)
    pltpu.sync_copy(row_h.at[pl.ds(base, EPW)], ridx)
    pltpu.sync_copy(col_h.at[pl.ds(base, EPW)], cidx)

    def body(i, carry):
        r = ridx[pl.ds(i * 16, 16)]
        cc = cidx[pl.ds(i * 16, 16)]
        dx = plsc.load_gather(px, [r]) - plsc.load_gather(px, [cc])
        dy = plsc.load_gather(py, [r]) - plsc.load_gather(py, [cc])
        dz = plsc.load_gather(pz, [r]) - plsc.load_gather(pz, [cc])
        d2[pl.ds(i * 16, 16)] = dx * dx + dy * dy + dz * dz
        return carry

    lax.fori_loop(0, EPW // 16, body, 0)
    pltpu.sync_copy(d2, d2_h.at[pl.ds(base, EPW)])


def _gather_body(x_h, col_h, xg_h, cidx, rows, sem, *, hoff, epw):
    c = lax.axis_index("c")
    s = lax.axis_index("s")
    base = (c * NS + s) * epw

    def body(w, carry):
        off = base + w * WIN
        pltpu.sync_copy(col_h.at[pl.ds(hoff + off, WIN)], cidx)
        pltpu.async_copy(x_h.at[cidx], rows, sem).wait()
        pltpu.sync_copy(rows, xg_h.at[pl.ds(off, WIN)])
        return carry

    lax.fori_loop(0, epw // WIN, body, 0)


def _scatter_body(msg_h, row_h, zero_h, agg_h, ridx, mrows, acc_sh, *, hoff,
                  epw):
    c = lax.axis_index("c")
    s = lax.axis_index("s")
    base = (c * NS + s) * epw
    pltpu.sync_copy(zero_h.at[pl.ds(s * SRP, SRP)], acc_sh.at[pl.ds(s * SRP, SRP)])

    @pl.when(s == 0)
    def _():
        pltpu.sync_copy(zero_h.at[pl.ds(NS * SRP, TAIL)],
                        acc_sh.at[pl.ds(NS * SRP, TAIL)])

    plsc.subcore_barrier()

    def body(w, carry):
        off = base + w * WIN
        pltpu.sync_copy(row_h.at[pl.ds(hoff + off, WIN)], ridx)
        pltpu.sync_copy(msg_h.at[pl.ds(off, WIN)], mrows)
        pltpu.sync_copy(mrows, acc_sh.at[ridx], add=True)
        return carry

    lax.fori_loop(0, epw // WIN, body, 0)
    plsc.subcore_barrier()
    pltpu.sync_copy(acc_sh.at[pl.ds(s * SRP, SRP)],
                    agg_h.at[pl.ds(c * N + s * SRP, SRP)])

    @pl.when(s == 0)
    def _():
        pltpu.sync_copy(acc_sh.at[pl.ds(NS * SRP, TAIL)],
                        agg_h.at[pl.ds(c * N + NS * SRP, TAIL)])


@functools.lru_cache(maxsize=None)
def _sc_kernels():
    mesh = plsc.VectorSubcoreMesh(core_axis_name="c", subcore_axis_name="s",
                                  num_cores=NC, num_subcores=NS)
    geom = pl.kernel(
        _geom_body,
        out_type=jax.ShapeDtypeStruct((E,), f32),
        mesh=mesh,
        compiler_params=pltpu.CompilerParams(needs_layout_passes=False),
        scratch_types=[
            pltpu.VMEM((N,), f32),
            pltpu.VMEM((N,), f32),
            pltpu.VMEM((N,), f32),
            pltpu.VMEM((EPW,), i32),
            pltpu.VMEM((EPW,), i32),
            pltpu.VMEM((EPW,), f32),
        ],
    )
    gathers = []
    scatters = []
    for hoff, eh in zip(HOFFS, EHS):
        epw = eh // NW
        gathers.append(pl.kernel(
            functools.partial(_gather_body, hoff=hoff, epw=epw),
            out_type=jax.ShapeDtypeStruct((eh, H), f32),
            mesh=mesh,
            scratch_types=[
                pltpu.VMEM((WIN,), i32),
                pltpu.VMEM((WIN, H), f32),
                pltpu.SemaphoreType.DMA,
            ],
        ))
        scatters.append(pl.kernel(
            functools.partial(_scatter_body, hoff=hoff, epw=epw),
            out_type=jax.ShapeDtypeStruct((NC * N, H), f32),
            mesh=mesh,
            scratch_types=[
                pltpu.VMEM((WIN,), i32),
                pltpu.VMEM((WIN, H), f32),
                pltpu.VMEM_SHARED((N, H), f32),
            ],
        ))
    return geom, gathers, scatters


def _embed_kernel(z_ref, emb_ref, w1n_ref, h_ref, x_ref):
    zb = z_ref[0]
    lanes = lax.broadcasted_iota(i32, (VP, 1), 0)
    oht = (zb == lanes).astype(f32)
    h = lax.dot_general(oht, emb_ref[...], (((0,), (0,)), ((), ())),
                        preferred_element_type=f32,
                        precision=lax.Precision.HIGHEST)
    h_ref[...] = h
    x_ref[...] = _dotd(h, w1n_ref[...])


def _ea_kernel(d2_ref, offs_ref, ea_ref, cc_ref):
    d2r = d2_ref[0]
    dr = jnp.sqrt(d2r + 1e-12)
    cc_ref[0] = (0.5 * (jnp.cos(dr * math.pi / CUT) + 1.0)
                 * (dr < CUT).astype(f32))
    d = dr.reshape(BE, 1)
    ea_ref[...] = jnp.exp(_COEFF * (d - offs_ref[...]) ** 2).astype(jnp.bfloat16)


def _filter_kernel(ea_ref, cc_ref, xg_ref, w1_ref, b1_ref, w2_ref, b2_ref,
                   msg_ref):
    t1 = _ssp(jnp.dot(ea_ref[...], w1_ref[...],
                      preferred_element_type=f32) + b1_ref[...])
    wf = _dotd(t1, w2_ref[...]) + b2_ref[...]
    cc = cc_ref[0].reshape(BE, 1)
    msg_ref[...] = xg_ref[...] * wf * cc


def _node_kernel(a0_ref, a1_ref, a2_ref, a3_ref, h_ref, w2_ref, b2_ref,
                 w_ref, b_ref, w1n_ref, hn_ref, xn_ref):
    agg = (a0_ref[0] + a1_ref[0]) + (a2_ref[0] + a3_ref[0])
    t = _ssp(_dotd(agg, w2_ref[...]) + b2_ref[...])
    x2 = _dotd(t, w_ref[...]) + b_ref[...]
    hn = h_ref[...] + x2
    hn_ref[...] = hn
    xn_ref[...] = _dotd(hn, w1n_ref[...])


def _pool_kernel(batch_ref, h_ref, o1w_ref, o1b_ref, o2w_ref, o2b_ref,
                 out_ref, acc):
    i = pl.program_id(0)

    @pl.when(i == 0)
    def _():
        acc[...] = jnp.zeros_like(acc)

    bb = batch_ref[0]
    seg = lax.broadcasted_iota(i32, (NG, 1), 0)
    oht = (bb == seg).astype(f32)
    acc[...] += lax.dot_general(oht, h_ref[...], (((1,), (0,)), ((), ())),
                                preferred_element_type=f32, precision=lax.Precision.HIGHEST)

    @pl.when(i == pl.num_programs(0) - 1)
    def _():
        t = _dotd(acc[...], o1w_ref[...]) + o1b_ref[...]
        t = t * (1.0 / (1.0 + jnp.exp(-t)))
        out_ref[...] = _dotd(t, o2w_ref[...]) + o2b_ref[...]


def _full_spec(shape):
    return pl.BlockSpec(shape, lambda *_: tuple(0 for _ in shape))


def _tc_embed(z3, embp, w1n):
    return pl.pallas_call(
        _embed_kernel,
        grid=(GRID_N,),
        in_specs=[
            pl.BlockSpec((1, 1, BN), lambda i: (i, 0, 0)),
            _full_spec((VP, H)),
            _full_spec((H, H)),
        ],
        out_specs=[
            pl.BlockSpec((BN, H), lambda i: (i, 0)),
            pl.BlockSpec((BN, H), lambda i: (i, 0)),
        ],
        out_shape=[
            jax.ShapeDtypeStruct((N, H), f32),
            jax.ShapeDtypeStruct((N, H), f32),
        ],
        compiler_params=pltpu.CompilerParams(
            dimension_semantics=("parallel",)),
    )(z3, embp, w1n)


def _tc_ea(d2g, offs, half):
    hblk = HOFFS[half] // BE
    eh = EHS[half]
    return pl.pallas_call(
        _ea_kernel,
        grid=(eh // BE,),
        in_specs=[
            pl.BlockSpec((1, 1, BE), lambda i: (i + hblk, 0, 0)),
            _full_spec((1, GP)),
        ],
        out_specs=[
            pl.BlockSpec((BE, GP), lambda i: (i, 0)),
            pl.BlockSpec((1, 1, BE), lambda i: (i, 0, 0)),
        ],
        out_shape=[
            jax.ShapeDtypeStruct((eh, GP), jnp.bfloat16),
            jax.ShapeDtypeStruct((eh // BE, 1, BE), f32),
        ],
        compiler_params=pltpu.CompilerParams(
            dimension_semantics=("parallel",)),
    )(d2g, offs)


def _tc_filter(ea, cc, xg, w1, b1, w2, b2, half):
    eh = EHS[half]
    return pl.pallas_call(
        _filter_kernel,
        grid=(eh // BE,),
        in_specs=[
            pl.BlockSpec((BE, GP), lambda i: (i, 0)),
            pl.BlockSpec((1, 1, BE), lambda i: (i, 0, 0)),
            pl.BlockSpec((BE, H), lambda i: (i, 0)),
            _full_spec((GP, H)),
            _full_spec((1, H)),
            _full_spec((H, H)),
            _full_spec((1, H)),
        ],
        out_specs=pl.BlockSpec((BE, H), lambda i: (i, 0)),
        out_shape=jax.ShapeDtypeStruct((eh, H), f32),
        compiler_params=pltpu.CompilerParams(
            dimension_semantics=("parallel",)),
    )(ea, cc, xg, w1, b1, w2, b2)


def _tc_node(agg1, agg2, h, w2, b2, w, b, w1n):
    return pl.pallas_call(
        _node_kernel,
        grid=(GRID_N,),
        in_specs=[
            pl.BlockSpec((1, BN, H), lambda i: (0, i, 0)),
            pl.BlockSpec((1, BN, H), lambda i: (1, i, 0)),
            pl.BlockSpec((1, BN, H), lambda i: (0, i, 0)),
            pl.BlockSpec((1, BN, H), lambda i: (1, i, 0)),
            pl.BlockSpec((BN, H), lambda i: (i, 0)),
            _full_spec((H, H)),
            _full_spec((1, H)),
            _full_spec((H, H)),
            _full_spec((1, H)),
            _full_spec((H, H)),
        ],
        out_specs=[
            pl.BlockSpec((BN, H), lambda i: (i, 0)),
            pl.BlockSpec((BN, H), lambda i: (i, 0)),
        ],
        out_shape=[
            jax.ShapeDtypeStruct((N, H), f32),
            jax.ShapeDtypeStruct((N, H), f32),
        ],
        compiler_params=pltpu.CompilerParams(
            dimension_semantics=("parallel",)),
    )(agg1, agg1, agg2, agg2, h, w2, b2, w, b, w1n)


def _tc_pool(batch3, h, o1w, o1b, o2w, o2b):
    return pl.pallas_call(
        _pool_kernel,
        grid=(GRID_N,),
        in_specs=[
            pl.BlockSpec((1, 1, BN), lambda i: (i, 0, 0)),
            pl.BlockSpec((BN, H), lambda i: (i, 0)),
            _full_spec((H, H)),
            _full_spec((1, H)),
            _full_spec((H, H)),
            _full_spec((1, H)),
        ],
        out_specs=pl.BlockSpec((NG, H), lambda i: (0, 0)),
        out_shape=jax.ShapeDtypeStruct((NG, H), f32),
        scratch_shapes=[pltpu.VMEM((NG, H), f32)],
        compiler_params=pltpu.CompilerParams(
            dimension_semantics=("arbitrary",)),
    )(batch3, h, o1w, o1b, o2w, o2b)


def kernel(z, pos, edge_index, batch, emb, mlp_w1, mlp_b1, mlp_w2, mlp_b2,
           lin1_w, lin2_w, lin2_b, lin_w, lin_b, out1_w, out1_b, out2_w,
           out2_b):
    row = edge_index[0].astype(i32)
    col = edge_index[1].astype(i32)
    px = pos[:, 0]
    py = pos[:, 1]
    pz = pos[:, 2]
    z3 = z.astype(i32).reshape(GRID_N, 1, BN)
    batch3 = batch.astype(i32).reshape(GRID_N, 1, BN)

    embp = jnp.pad(emb, ((0, VP - emb.shape[0]), (0, 0)))
    w1p = jnp.pad(mlp_w1, ((0, 0), (0, GP - G), (0, 0))).astype(jnp.bfloat16)
    offs = jnp.linspace(0.0, CUT, G, dtype=f32)
    offsp = jnp.pad(offs, (0, GP - G), constant_values=1e6).reshape(1, GP)
    o1wp = jnp.pad(out1_w, ((0, 0), (0, H - out1_w.shape[1])))
    o1bp = jnp.pad(out1_b, (0, H - out1_b.shape[0])).reshape(1, H)
    o2wp = jnp.pad(out2_w, ((0, H - out2_w.shape[0]), (0, H - out2_w.shape[1])))
    o2bp = jnp.pad(out2_b, (0, H - out2_b.shape[0])).reshape(1, H)
    zeros_nh = jnp.zeros((N, H), f32)
    _geom, _gather_rows, _scatter_add = _sc_kernels()

    d2 = _geom(px, py, pz, row, col)
    d2g = d2.reshape(GRID_E, 1, BE)

    h, x = _tc_embed(z3, embp, lin1_w[0])

    eas = [_tc_ea(d2g, offsp, 0), _tc_ea(d2g, offsp, 1)]

    for l in range(L):
        b1l = mlp_b1[l].reshape(1, H)
        b2l = mlp_b2[l].reshape(1, H)
        xg1 = _gather_rows[0](x, col)
        msg1 = _tc_filter(eas[0][0], eas[0][1], xg1, w1p[l], b1l,
                          mlp_w2[l], b2l, 0)
        xg2 = _gather_rows[1](x, col)
        agg1 = _scatter_add[0](msg1, row, zeros_nh)
        msg2 = _tc_filter(eas[1][0], eas[1][1], xg2, w1p[l], b1l,
                          mlp_w2[l], b2l, 1)
        agg2 = _scatter_add[1](msg2, row, zeros_nh)
        w1n = lin1_w[(l + 1) % L]
        h, x = _tc_node(agg1.reshape(NC, N, H), agg2.reshape(NC, N, H), h,
                        lin2_w[l], lin2_b[l].reshape(1, H), lin_w[l],
                        lin_b[l].reshape(1, H), w1n)

    outp = _tc_pool(batch3, h, o1wp, o1bp, o2wp, o2bp)
    return outp[:, :1]


# speedup vs baseline: 4.7639x; 1.4747x over previous
import functools
import math

import jax
import jax.numpy as jnp
from jax import lax
from jax.experimental import pallas as pl
from jax.experimental.pallas import tpu as pltpu
from jax.experimental.pallas import tpu_sc as plsc

N = 10000
E = 320000
H = 128
G = 50
GP = 64
NG = 16
L = 4
CUT = 10.0
VP = 128

NC = 2
NS = 16
NW = NC * NS
EPW = E // NW
WIN = 80
NWIN = EPW // WIN
SRP = 624
TAIL = N - NS * SRP

EH1 = 192000
EH2 = E - EH1
EHS = (EH1, EH2)
HOFFS = (0, EH1)

BE = 2560
GRID_E = E // BE
BN = 2000
GRID_N = N // BN

_SPACING = CUT / (G - 1)
_COEFF = -0.5 / (_SPACING * _SPACING)

f32 = jnp.float32
i32 = jnp.int32


def _dotd(a, b):
    return jnp.dot(a.astype(jnp.bfloat16), b.astype(jnp.bfloat16),
                   preferred_element_type=f32)


def _ssp(x):
    return jnp.maximum(x, 0.0) + jnp.log1p(jnp.exp(-jnp.abs(x))) - math.log(2.0)


def _geom_body(px_h, py_h, pz_h, row_h, col_h, d2_h, px, py, pz, ridx, cidx, d2):
    c = lax.axis_index("c")
    s = lax.axis_index("s")
    base = (c * NS + s) * EPW
    pltpu.sync_copy(px_h, px)
    pltpu.sync_copy(py_h, py)
    pltpu.sync_copy(pz_h, pz)
    pltpu.sync_copy(row_h.at[pl.ds(base, EPW)], ridx)
    pltpu.sync_copy(col_h.at[pl.ds(base, EPW)], cidx)

    def body(i, carry):
        r = ridx[pl.ds(i * 16, 16)]
        cc = cidx[pl.ds(i * 16, 16)]
        dx = plsc.load_gather(px, [r]) - plsc.load_gather(px, [cc])
        dy = plsc.load_gather(py, [r]) - plsc.load_gather(py, [cc])
        dz = plsc.load_gather(pz, [r]) - plsc.load_gather(pz, [cc])
        d2[pl.ds(i * 16, 16)] = dx * dx + dy * dy + dz * dz
        return carry

    lax.fori_loop(0, EPW // 16, body, 0)
    pltpu.sync_copy(d2, d2_h.at[pl.ds(base, EPW)])


def _gather_body(x_h, col_h, xg_h, cidx, rows0, rows1, sem0, sem1, *, hoff,
                 epw):
    c = lax.axis_index("c")
    s = lax.axis_index("s")
    base = (c * NS + s) * epw
    nwin = epw // WIN
    rows = (rows0, rows1)
    sems = (sem0, sem1)

    pltpu.sync_copy(col_h.at[pl.ds(hoff + base, epw)], cidx)

    for b in range(2):
        pltpu.async_copy(x_h.at[cidx.at[pl.ds(b * WIN, WIN)]], rows[b],
                         sems[b])

    def body(w, carry):
        for b in range(2):
            @pl.when(w * 2 + b < nwin)
            def _():
                ww = w * 2 + b
                pltpu.make_async_copy(x_h.at[cidx.at[pl.ds(0, WIN)]],
                                      rows[b], sems[b]).wait()
                pltpu.sync_copy(rows[b], xg_h.at[pl.ds(base + ww * WIN, WIN)])

                @pl.when(ww + 2 < nwin)
                def _():
                    pltpu.async_copy(
                        x_h.at[cidx.at[pl.ds((ww + 2) * WIN, WIN)]],
                        rows[b], sems[b])
        return carry

    lax.fori_loop(0, (nwin + 1) // 2, body, 0)


def _scatter_body(msg_h, row_h, zero_h, agg_h, ridx0, ridx1, mrows0,
                  mrows1, sem0, sem1, isem0, isem1, acc_sh, *, hoff, epw):
    c = lax.axis_index("c")
    s = lax.axis_index("s")
    base = (c * NS + s) * epw
    nwin = epw // WIN
    ridx = (ridx0, ridx1)
    mrows = (mrows0, mrows1)
    sems = (sem0, sem1)
    isems = (isem0, isem1)
    pltpu.sync_copy(zero_h.at[pl.ds(s * SRP, SRP)], acc_sh.at[pl.ds(s * SRP, SRP)])

    @pl.when(s == 0)
    def _():
        pltpu.sync_copy(zero_h.at[pl.ds(NS * SRP, TAIL)],
                        acc_sh.at[pl.ds(NS * SRP, TAIL)])

    plsc.subcore_barrier()

    for b in range(2):
        pltpu.async_copy(msg_h.at[pl.ds(base + b * WIN, WIN)], mrows[b],
                         sems[b])
        pltpu.async_copy(row_h.at[pl.ds(hoff + base + b * WIN, WIN)],
                         ridx[b], isems[b])

    def body(w, carry):
        for b in range(2):
            @pl.when(w * 2 + b < nwin)
            def _():
                ww = w * 2 + b
                pltpu.make_async_copy(msg_h.at[pl.ds(0, WIN)], mrows[b],
                                      sems[b]).wait()
                pltpu.make_async_copy(row_h.at[pl.ds(0, WIN)], ridx[b],
                                      isems[b]).wait()
                pltpu.sync_copy(mrows[b], acc_sh.at[ridx[b]], add=True)

                @pl.when(ww + 2 < nwin)
                def _():
                    pltpu.async_copy(
                        msg_h.at[pl.ds(base + (ww + 2) * WIN, WIN)],
                        mrows[b], sems[b])
                    pltpu.async_copy(
                        row_h.at[pl.ds(hoff + base + (ww + 2) * WIN, WIN)],
                        ridx[b], isems[b])
        return carry

    lax.fori_loop(0, (nwin + 1) // 2, body, 0)
    plsc.subcore_barrier()
    pltpu.sync_copy(acc_sh.at[pl.ds(s * SRP, SRP)],
                    agg_h.at[pl.ds(c * N + s * SRP, SRP)])

    @pl.when(s == 0)
    def _():
        pltpu.sync_copy(acc_sh.at[pl.ds(NS * SRP, TAIL)],
                        agg_h.at[pl.ds(c * N + NS * SRP, TAIL)])


@functools.lru_cache(maxsize=None)
def _sc_kernels():
    mesh = plsc.VectorSubcoreMesh(core_axis_name="c", subcore_axis_name="s",
                                  num_cores=NC, num_subcores=NS)
    geom = pl.kernel(
        _geom_body,
        out_type=jax.ShapeDtypeStruct((E,), f32),
        mesh=mesh,
        compiler_params=pltpu.CompilerParams(needs_layout_passes=False),
        scratch_types=[
            pltpu.VMEM((N,), f32),
            pltpu.VMEM((N,), f32),
            pltpu.VMEM((N,), f32),
            pltpu.VMEM((EPW,), i32),
            pltpu.VMEM((EPW,), i32),
            pltpu.VMEM((EPW,), f32),
        ],
    )
    gathers = []
    scatters = []
    for hoff, eh in zip(HOFFS, EHS):
        epw = eh // NW
        nwin = epw // WIN
        gathers.append(pl.kernel(
            functools.partial(_gather_body, hoff=hoff, epw=epw),
            out_type=jax.ShapeDtypeStruct((eh, H), f32),
            mesh=mesh,
            scratch_types=[
                pltpu.VMEM((epw,), i32),
                pltpu.VMEM((WIN, H), f32),
                pltpu.VMEM((WIN, H), f32),
                pltpu.SemaphoreType.DMA,
                pltpu.SemaphoreType.DMA,
            ],
        ))
        scatters.append(pl.kernel(
            functools.partial(_scatter_body, hoff=hoff, epw=epw),
            out_type=jax.ShapeDtypeStruct((NC * N, H), f32),
            mesh=mesh,
            scratch_types=[
                pltpu.VMEM((WIN,), i32),
                pltpu.VMEM((WIN,), i32),
                pltpu.VMEM((WIN, H), f32),
                pltpu.VMEM((WIN, H), f32),
                pltpu.SemaphoreType.DMA,
                pltpu.SemaphoreType.DMA,
                pltpu.SemaphoreType.DMA,
                pltpu.SemaphoreType.DMA,
                pltpu.VMEM_SHARED((N, H), f32),
            ],
        ))
    return geom, gathers, scatters


def _embed_kernel(z_ref, emb_ref, w1n_ref, h_ref, x_ref):
    zb = z_ref[0]
    lanes = lax.broadcasted_iota(i32, (VP, 1), 0)
    oht = (zb == lanes).astype(f32)
    h = lax.dot_general(oht, emb_ref[...], (((0,), (0,)), ((), ())),
                        preferred_element_type=f32,
                        precision=lax.Precision.HIGHEST)
    h_ref[...] = h
    x_ref[...] = _dotd(h, w1n_ref[...])


def _ea_kernel(d2_ref, offs_ref, ea_ref, cc_ref):
    d2r = d2_ref[0]
    dr = jnp.sqrt(d2r + 1e-12)
    cc_ref[0] = (0.5 * (jnp.cos(dr * math.pi / CUT) + 1.0)
                 * (dr < CUT).astype(f32))
    d = dr.reshape(BE, 1)
    ea_ref[...] = jnp.exp(_COEFF * (d - offs_ref[...]) ** 2).astype(jnp.bfloat16)


def _filter_kernel(ea_ref, cc_ref, xg_ref, w1_ref, b1_ref, w2_ref, b2_ref,
                   msg_ref):
    t1 = _ssp(jnp.dot(ea_ref[...], w1_ref[...],
                      preferred_element_type=f32) + b1_ref[...])
    wf = _dotd(t1, w2_ref[...]) + b2_ref[...]
    cc = cc_ref[0].reshape(BE, 1)
    msg_ref[...] = xg_ref[...] * wf * cc


def _node_kernel(a0_ref, a1_ref, a2_ref, a3_ref, h_ref, w2_ref, b2_ref,
                 w_ref, b_ref, w1n_ref, hn_ref, xn_ref):
    agg = (a0_ref[0] + a1_ref[0]) + (a2_ref[0] + a3_ref[0])
    t = _ssp(_dotd(agg, w2_ref[...]) + b2_ref[...])
    x2 = _dotd(t, w_ref[...]) + b_ref[...]
    hn = h_ref[...] + x2
    hn_ref[...] = hn
    xn_ref[...] = _dotd(hn, w1n_ref[...])


def _pool_kernel(batch_ref, h_ref, o1w_ref, o1b_ref, o2w_ref, o2b_ref,
                 out_ref, acc):
    i = pl.program_id(0)

    @pl.when(i == 0)
    def _():
        acc[...] = jnp.zeros_like(acc)

    bb = batch_ref[0]
    seg = lax.broadcasted_iota(i32, (NG, 1), 0)
    oht = (bb == seg).astype(f32)
    acc[...] += lax.dot_general(oht, h_ref[...], (((1,), (0,)), ((), ())),
                                preferred_element_type=f32, precision=lax.Precision.HIGHEST)

    @pl.when(i == pl.num_programs(0) - 1)
    def _():
        t = _dotd(acc[...], o1w_ref[...]) + o1b_ref[...]
        t = t * (1.0 / (1.0 + jnp.exp(-t)))
        out_ref[...] = _dotd(t, o2w_ref[...]) + o2b_ref[...]


def _full_spec(shape):
    return pl.BlockSpec(shape, lambda *_: tuple(0 for _ in shape))


def _tc_embed(z3, embp, w1n):
    return pl.pallas_call(
        _embed_kernel,
        grid=(GRID_N,),
        in_specs=[
            pl.BlockSpec((1, 1, BN), lambda i: (i, 0, 0)),
            _full_spec((VP, H)),
            _full_spec((H, H)),
        ],
        out_specs=[
            pl.BlockSpec((BN, H), lambda i: (i, 0)),
            pl.BlockSpec((BN, H), lambda i: (i, 0)),
        ],
        out_shape=[
            jax.ShapeDtypeStruct((N, H), f32),
            jax.ShapeDtypeStruct((N, H), f32),
        ],
        compiler_params=pltpu.CompilerParams(
            dimension_semantics=("parallel",)),
    )(z3, embp, w1n)


def _tc_ea(d2g, offs, half):
    hblk = HOFFS[half] // BE
    eh = EHS[half]
    return pl.pallas_call(
        _ea_kernel,
        grid=(eh // BE,),
        in_specs=[
            pl.BlockSpec((1, 1, BE), lambda i: (i + hblk, 0, 0)),
            _full_spec((1, GP)),
        ],
        out_specs=[
            pl.BlockSpec((BE, GP), lambda i: (i, 0)),
            pl.BlockSpec((1, 1, BE), lambda i: (i, 0, 0)),
        ],
        out_shape=[
            jax.ShapeDtypeStruct((eh, GP), jnp.bfloat16),
            jax.ShapeDtypeStruct((eh // BE, 1, BE), f32),
        ],
        compiler_params=pltpu.CompilerParams(
            dimension_semantics=("parallel",)),
    )(d2g, offs)


def _tc_filter(ea, cc, xg, w1, b1, w2, b2, half):
    eh = EHS[half]
    return pl.pallas_call(
        _filter_kernel,
        grid=(eh // BE,),
        in_specs=[
            pl.BlockSpec((BE, GP), lambda i: (i, 0)),
            pl.BlockSpec((1, 1, BE), lambda i: (i, 0, 0)),
            pl.BlockSpec((BE, H), lambda i: (i, 0)),
            _full_spec((GP, H)),
            _full_spec((1, H)),
            _full_spec((H, H)),
            _full_spec((1, H)),
        ],
        out_specs=pl.BlockSpec((BE, H), lambda i: (i, 0)),
        out_shape=jax.ShapeDtypeStruct((eh, H), f32),
        compiler_params=pltpu.CompilerParams(
            dimension_semantics=("parallel",)),
    )(ea, cc, xg, w1, b1, w2, b2)


def _tc_node(agg1, agg2, h, w2, b2, w, b, w1n):
    return pl.pallas_call(
        _node_kernel,
        grid=(GRID_N,),
        in_specs=[
            pl.BlockSpec((1, BN, H), lambda i: (0, i, 0)),
            pl.BlockSpec((1, BN, H), lambda i: (1, i, 0)),
            pl.BlockSpec((1, BN, H), lambda i: (0, i, 0)),
            pl.BlockSpec((1, BN, H), lambda i: (1, i, 0)),
            pl.BlockSpec((BN, H), lambda i: (i, 0)),
            _full_spec((H, H)),
            _full_spec((1, H)),
            _full_spec((H, H)),
            _full_spec((1, H)),
            _full_spec((H, H)),
        ],
        out_specs=[
            pl.BlockSpec((BN, H), lambda i: (i, 0)),
            pl.BlockSpec((BN, H), lambda i: (i, 0)),
        ],
        out_shape=[
            jax.ShapeDtypeStruct((N, H), f32),
            jax.ShapeDtypeStruct((N, H), f32),
        ],
        compiler_params=pltpu.CompilerParams(
            dimension_semantics=("parallel",)),
    )(agg1, agg1, agg2, agg2, h, w2, b2, w, b, w1n)


def _tc_pool(batch3, h, o1w, o1b, o2w, o2b):
    return pl.pallas_call(
        _pool_kernel,
        grid=(GRID_N,),
        in_specs=[
            pl.BlockSpec((1, 1, BN), lambda i: (i, 0, 0)),
            pl.BlockSpec((BN, H), lambda i: (i, 0)),
            _full_spec((H, H)),
            _full_spec((1, H)),
            _full_spec((H, H)),
            _full_spec((1, H)),
        ],
        out_specs=pl.BlockSpec((NG, H), lambda i: (0, 0)),
        out_shape=jax.ShapeDtypeStruct((NG, H), f32),
        scratch_shapes=[pltpu.VMEM((NG, H), f32)],
        compiler_params=pltpu.CompilerParams(
            dimension_semantics=("arbitrary",)),
    )(batch3, h, o1w, o1b, o2w, o2b)


def kernel(z, pos, edge_index, batch, emb, mlp_w1, mlp_b1, mlp_w2, mlp_b2,
           lin1_w, lin2_w, lin2_b, lin_w, lin_b, out1_w, out1_b, out2_w,
           out2_b):
    row = edge_index[0].astype(i32)
    col = edge_index[1].astype(i32)
    px = pos[:, 0]
    py = pos[:, 1]
    pz = pos[:, 2]
    z3 = z.astype(i32).reshape(GRID_N, 1, BN)
    batch3 = batch.astype(i32).reshape(GRID_N, 1, BN)

    embp = jnp.pad(emb, ((0, VP - emb.shape[0]), (0, 0)))
    w1p = jnp.pad(mlp_w1, ((0, 0), (0, GP - G), (0, 0))).astype(jnp.bfloat16)
    offs = jnp.linspace(0.0, CUT, G, dtype=f32)
    offsp = jnp.pad(offs, (0, GP - G), constant_values=1e6).reshape(1, GP)
    o1wp = jnp.pad(out1_w, ((0, 0), (0, H - out1_w.shape[1])))
    o1bp = jnp.pad(out1_b, (0, H - out1_b.shape[0])).reshape(1, H)
    o2wp = jnp.pad(out2_w, ((0, H - out2_w.shape[0]), (0, H - out2_w.shape[1])))
    o2bp = jnp.pad(out2_b, (0, H - out2_b.shape[0])).reshape(1, H)
    zeros_nh = jnp.zeros((N, H), f32)
    _geom, _gather_rows, _scatter_add = _sc_kernels()

    d2 = _geom(px, py, pz, row, col)
    d2g = d2.reshape(GRID_E, 1, BE)

    h, x = _tc_embed(z3, embp, lin1_w[0])

    eas = [_tc_ea(d2g, offsp, 0), _tc_ea(d2g, offsp, 1)]

    for l in range(L):
        b1l = mlp_b1[l].reshape(1, H)
        b2l = mlp_b2[l].reshape(1, H)
        xg1 = _gather_rows[0](x, col)
        msg1 = _tc_filter(eas[0][0], eas[0][1], xg1, w1p[l], b1l,
                          mlp_w2[l], b2l, 0)
        xg2 = _gather_rows[1](x, col)
        agg1 = _scatter_add[0](msg1, row, zeros_nh)
        msg2 = _tc_filter(eas[1][0], eas[1][1], xg2, w1p[l], b1l,
                          mlp_w2[l], b2l, 1)
        agg2 = _scatter_add[1](msg2, row, zeros_nh)
        w1n = lin1_w[(l + 1) % L]
        h, x = _tc_node(agg1.reshape(NC, N, H), agg2.reshape(NC, N, H), h,
                        lin2_w[l], lin2_b[l].reshape(1, H), lin_w[l],
                        lin_b[l].reshape(1, H), w1n)

    outp = _tc_pool(batch3, h, o1wp, o1bp, o2wp, o2bp)
    return outp[:, :1]


# speedup vs baseline: 4.9186x; 1.0325x over previous
import functools
import math

import jax
import jax.numpy as jnp
from jax import lax
from jax.experimental import pallas as pl
from jax.experimental.pallas import tpu as pltpu
from jax.experimental.pallas import tpu_sc as plsc

N = 10000
E = 320000
H = 128
G = 50
GP = 64
NG = 16
L = 4
CUT = 10.0
VP = 128

NC = 2
NS = 16
NW = NC * NS
EPW = E // NW
WIN = 80
NWIN = EPW // WIN
SRP = 624
TAIL = N - NS * SRP

EHS = (104960, 104960, 110080)
HOFFS = (0, 104960, 209920)
NCHUNK = len(EHS)

BE = 2560
GRID_E = E // BE
BN = 2000
GRID_N = N // BN

_SPACING = CUT / (G - 1)
_COEFF = -0.5 / (_SPACING * _SPACING)

f32 = jnp.float32
i32 = jnp.int32


def _dotd(a, b):
    return jnp.dot(a.astype(jnp.bfloat16), b.astype(jnp.bfloat16),
                   preferred_element_type=f32)


def _ssp(x):
    return jnp.maximum(x, 0.0) + jnp.log1p(jnp.exp(-jnp.abs(x))) - math.log(2.0)


def _geom_body(px_h, py_h, pz_h, row_h, col_h, d2_h, px, py, pz, ridx, cidx, d2):
    c = lax.axis_index("c")
    s = lax.axis_index("s")
    base = (c * NS + s) * EPW
    pltpu.sync_copy(px_h, px)
    pltpu.sync_copy(py_h, py)
    pltpu.sync_copy(pz_h, pz)
    pltpu.sync_copy(row_h.at[pl.ds(base, EPW)], ridx)
    pltpu.sync_copy(col_h.at[pl.ds(base, EPW)], cidx)

    def body(i, carry):
        r = ridx[pl.ds(i * 16, 16)]
        cc = cidx[pl.ds(i * 16, 16)]
        dx = plsc.load_gather(px, [r]) - plsc.load_gather(px, [cc])
        dy = plsc.load_gather(py, [r]) - plsc.load_gather(py, [cc])
        dz = plsc.load_gather(pz, [r]) - plsc.load_gather(pz, [cc])
        d2[pl.ds(i * 16, 16)] = dx * dx + dy * dy + dz * dz
        return carry

    lax.fori_loop(0, EPW // 16, body, 0)
    pltpu.sync_copy(d2, d2_h.at[pl.ds(base, EPW)])


def _gather_body(x_h, col_h, xg_h, cidx, rows0, rows1, sem0, sem1, *, hoff,
                 epw):
    c = lax.axis_index("c")
    s = lax.axis_index("s")
    base = (c * NS + s) * epw
    nwin = epw // WIN
    rows = (rows0, rows1)
    sems = (sem0, sem1)

    pltpu.sync_copy(col_h.at[pl.ds(hoff + base, epw)], cidx)

    for b in range(2):
        pltpu.async_copy(x_h.at[cidx.at[pl.ds(b * WIN, WIN)]], rows[b],
                         sems[b])

    def body(w, carry):
        for b in range(2):
            @pl.when(w * 2 + b < nwin)
            def _():
                ww = w * 2 + b
                pltpu.make_async_copy(x_h.at[cidx.at[pl.ds(0, WIN)]],
                                      rows[b], sems[b]).wait()
                pltpu.sync_copy(rows[b], xg_h.at[pl.ds(base + ww * WIN, WIN)])

                @pl.when(ww + 2 < nwin)
                def _():
                    pltpu.async_copy(
                        x_h.at[cidx.at[pl.ds((ww + 2) * WIN, WIN)]],
                        rows[b], sems[b])
        return carry

    lax.fori_loop(0, (nwin + 1) // 2, body, 0)


def _scatter_body(msg_h, row_h, zero_h, agg_h, ridx0, ridx1, mrows0,
                  mrows1, sem0, sem1, isem0, isem1, acc_sh, *, hoff, epw):
    c = lax.axis_index("c")
    s = lax.axis_index("s")
    base = (c * NS + s) * epw
    nwin = epw // WIN
    ridx = (ridx0, ridx1)
    mrows = (mrows0, mrows1)
    sems = (sem0, sem1)
    isems = (isem0, isem1)
    pltpu.sync_copy(zero_h.at[pl.ds(s * SRP, SRP)], acc_sh.at[pl.ds(s * SRP, SRP)])

    @pl.when(s == 0)
    def _():
        pltpu.sync_copy(zero_h.at[pl.ds(NS * SRP, TAIL)],
                        acc_sh.at[pl.ds(NS * SRP, TAIL)])

    plsc.subcore_barrier()

    for b in range(2):
        pltpu.async_copy(msg_h.at[pl.ds(base + b * WIN, WIN)], mrows[b],
                         sems[b])
        pltpu.async_copy(row_h.at[pl.ds(hoff + base + b * WIN, WIN)],
                         ridx[b], isems[b])

    def body(w, carry):
        for b in range(2):
            @pl.when(w * 2 + b < nwin)
            def _():
                ww = w * 2 + b
                pltpu.make_async_copy(msg_h.at[pl.ds(0, WIN)], mrows[b],
                                      sems[b]).wait()
                pltpu.make_async_copy(row_h.at[pl.ds(0, WIN)], ridx[b],
                                      isems[b]).wait()
                pltpu.sync_copy(mrows[b], acc_sh.at[ridx[b]], add=True)

                @pl.when(ww + 2 < nwin)
                def _():
                    pltpu.async_copy(
                        msg_h.at[pl.ds(base + (ww + 2) * WIN, WIN)],
                        mrows[b], sems[b])
                    pltpu.async_copy(
                        row_h.at[pl.ds(hoff + base + (ww + 2) * WIN, WIN)],
                        ridx[b], isems[b])
        return carry

    lax.fori_loop(0, (nwin + 1) // 2, body, 0)
    plsc.subcore_barrier()
    pltpu.sync_copy(acc_sh.at[pl.ds(s * SRP, SRP)],
                    agg_h.at[pl.ds(c * N + s * SRP, SRP)])

    @pl.when(s == 0)
    def _():
        pltpu.sync_copy(acc_sh.at[pl.ds(NS * SRP, TAIL)],
                        agg_h.at[pl.ds(c * N + NS * SRP, TAIL)])


@functools.lru_cache(maxsize=None)
def _sc_kernels():
    mesh = plsc.VectorSubcoreMesh(core_axis_name="c", subcore_axis_name="s",
                                  num_cores=NC, num_subcores=NS)
    geom = pl.kernel(
        _geom_body,
        out_type=jax.ShapeDtypeStruct((E,), f32),
        mesh=mesh,
        compiler_params=pltpu.CompilerParams(needs_layout_passes=False),
        scratch_types=[
            pltpu.VMEM((N,), f32),
            pltpu.VMEM((N,), f32),
            pltpu.VMEM((N,), f32),
            pltpu.VMEM((EPW,), i32),
            pltpu.VMEM((EPW,), i32),
            pltpu.VMEM((EPW,), f32),
        ],
    )
    gathers = []
    scatters = []
    for hoff, eh in zip(HOFFS, EHS):
        epw = eh // NW
        nwin = epw // WIN
        gathers.append(pl.kernel(
            functools.partial(_gather_body, hoff=hoff, epw=epw),
            out_type=jax.ShapeDtypeStruct((eh, H), f32),
            mesh=mesh,
            scratch_types=[
                pltpu.VMEM((epw,), i32),
                pltpu.VMEM((WIN, H), f32),
                pltpu.VMEM((WIN, H), f32),
                pltpu.SemaphoreType.DMA,
                pltpu.SemaphoreType.DMA,
            ],
        ))
        scatters.append(pl.kernel(
            functools.partial(_scatter_body, hoff=hoff, epw=epw),
            out_type=jax.ShapeDtypeStruct((NC * N, H), f32),
            mesh=mesh,
            scratch_types=[
                pltpu.VMEM((WIN,), i32),
                pltpu.VMEM((WIN,), i32),
                pltpu.VMEM((WIN, H), f32),
                pltpu.VMEM((WIN, H), f32),
                pltpu.SemaphoreType.DMA,
                pltpu.SemaphoreType.DMA,
                pltpu.SemaphoreType.DMA,
                pltpu.SemaphoreType.DMA,
                pltpu.VMEM_SHARED((N, H), f32),
            ],
        ))
    return geom, gathers, scatters


def _embed_kernel(z_ref, emb_ref, w1n_ref, h_ref, x_ref):
    zb = z_ref[0]
    lanes = lax.broadcasted_iota(i32, (VP, 1), 0)
    oht = (zb == lanes).astype(f32)
    h = lax.dot_general(oht, emb_ref[...], (((0,), (0,)), ((), ())),
                        preferred_element_type=f32,
                        precision=lax.Precision.HIGHEST)
    h_ref[...] = h
    x_ref[...] = _dotd(h, w1n_ref[...])


def _ea_kernel(d2_ref, offs_ref, ea_ref, cc_ref):
    d2r = d2_ref[0]
    dr = jnp.sqrt(d2r + 1e-12)
    cc_ref[0] = (0.5 * (jnp.cos(dr * math.pi / CUT) + 1.0)
                 * (dr < CUT).astype(f32))
    d = dr.reshape(BE, 1)
    ea_ref[...] = jnp.exp(_COEFF * (d - offs_ref[...]) ** 2).astype(jnp.bfloat16)


def _filter_kernel(ea_ref, cc_ref, xg_ref, w1_ref, b1_ref, w2_ref, b2_ref,
                   msg_ref):
    t1 = _ssp(jnp.dot(ea_ref[...], w1_ref[...],
                      preferred_element_type=f32) + b1_ref[...])
    wf = _dotd(t1, w2_ref[...]) + b2_ref[...]
    cc = cc_ref[0].reshape(BE, 1)
    msg_ref[...] = xg_ref[...] * wf * cc


def _node_kernel(a0_ref, a1_ref, a2_ref, a3_ref, a4_ref, a5_ref, h_ref,
                 w2_ref, b2_ref, w_ref, b_ref, w1n_ref, hn_ref, xn_ref):
    agg = ((a0_ref[0] + a1_ref[0]) + (a2_ref[0] + a3_ref[0])
           + (a4_ref[0] + a5_ref[0]))
    t = _ssp(_dotd(agg, w2_ref[...]) + b2_ref[...])
    x2 = _dotd(t, w_ref[...]) + b_ref[...]
    hn = h_ref[...] + x2
    hn_ref[...] = hn
    xn_ref[...] = _dotd(hn, w1n_ref[...])


def _pool_kernel(batch_ref, h_ref, o1w_ref, o1b_ref, o2w_ref, o2b_ref,
                 out_ref, acc):
    i = pl.program_id(0)

    @pl.when(i == 0)
    def _():
        acc[...] = jnp.zeros_like(acc)

    bb = batch_ref[0]
    seg = lax.broadcasted_iota(i32, (NG, 1), 0)
    oht = (bb == seg).astype(f32)
    acc[...] += lax.dot_general(oht, h_ref[...], (((1,), (0,)), ((), ())),
                                preferred_element_type=f32, precision=lax.Precision.HIGHEST)

    @pl.when(i == pl.num_programs(0) - 1)
    def _():
        t = _dotd(acc[...], o1w_ref[...]) + o1b_ref[...]
        t = t * (1.0 / (1.0 + jnp.exp(-t)))
        out_ref[...] = _dotd(t, o2w_ref[...]) + o2b_ref[...]


def _full_spec(shape):
    return pl.BlockSpec(shape, lambda *_: tuple(0 for _ in shape))


def _tc_embed(z3, embp, w1n):
    return pl.pallas_call(
        _embed_kernel,
        grid=(GRID_N,),
        in_specs=[
            pl.BlockSpec((1, 1, BN), lambda i: (i, 0, 0)),
            _full_spec((VP, H)),
            _full_spec((H, H)),
        ],
        out_specs=[
            pl.BlockSpec((BN, H), lambda i: (i, 0)),
            pl.BlockSpec((BN, H), lambda i: (i, 0)),
        ],
        out_shape=[
            jax.ShapeDtypeStruct((N, H), f32),
            jax.ShapeDtypeStruct((N, H), f32),
        ],
        compiler_params=pltpu.CompilerParams(
            dimension_semantics=("parallel",)),
    )(z3, embp, w1n)


def _tc_ea(d2g, offs, half):
    hblk = HOFFS[half] // BE
    eh = EHS[half]
    return pl.pallas_call(
        _ea_kernel,
        grid=(eh // BE,),
        in_specs=[
            pl.BlockSpec((1, 1, BE), lambda i: (i + hblk, 0, 0)),
            _full_spec((1, GP)),
        ],
        out_specs=[
            pl.BlockSpec((BE, GP), lambda i: (i, 0)),
            pl.BlockSpec((1, 1, BE), lambda i: (i, 0, 0)),
        ],
        out_shape=[
            jax.ShapeDtypeStruct((eh, GP), jnp.bfloat16),
            jax.ShapeDtypeStruct((eh // BE, 1, BE), f32),
        ],
        compiler_params=pltpu.CompilerParams(
            dimension_semantics=("parallel",)),
    )(d2g, offs)


def _tc_filter(ea, cc, xg, w1, b1, w2, b2, half):
    eh = EHS[half]
    return pl.pallas_call(
        _filter_kernel,
        grid=(eh // BE,),
        in_specs=[
            pl.BlockSpec((BE, GP), lambda i: (i, 0)),
            pl.BlockSpec((1, 1, BE), lambda i: (i, 0, 0)),
            pl.BlockSpec((BE, H), lambda i: (i, 0)),
            _full_spec((GP, H)),
            _full_spec((1, H)),
            _full_spec((H, H)),
            _full_spec((1, H)),
        ],
        out_specs=pl.BlockSpec((BE, H), lambda i: (i, 0)),
        out_shape=jax.ShapeDtypeStruct((eh, H), f32),
        compiler_params=pltpu.CompilerParams(
            dimension_semantics=("parallel",)),
    )(ea, cc, xg, w1, b1, w2, b2)


def _tc_node(aggs, h, w2, b2, w, b, w1n):
    return pl.pallas_call(
        _node_kernel,
        grid=(GRID_N,),
        in_specs=[
            pl.BlockSpec((1, BN, H), lambda i: (0, i, 0)),
            pl.BlockSpec((1, BN, H), lambda i: (1, i, 0)),
            pl.BlockSpec((1, BN, H), lambda i: (0, i, 0)),
            pl.BlockSpec((1, BN, H), lambda i: (1, i, 0)),
            pl.BlockSpec((1, BN, H), lambda i: (0, i, 0)),
            pl.BlockSpec((1, BN, H), lambda i: (1, i, 0)),
            pl.BlockSpec((BN, H), lambda i: (i, 0)),
            _full_spec((H, H)),
            _full_spec((1, H)),
            _full_spec((H, H)),
            _full_spec((1, H)),
            _full_spec((H, H)),
        ],
        out_specs=[
            pl.BlockSpec((BN, H), lambda i: (i, 0)),
            pl.BlockSpec((BN, H), lambda i: (i, 0)),
        ],
        out_shape=[
            jax.ShapeDtypeStruct((N, H), f32),
            jax.ShapeDtypeStruct((N, H), f32),
        ],
        compiler_params=pltpu.CompilerParams(
            dimension_semantics=("parallel",)),
    )(aggs[0], aggs[0], aggs[1], aggs[1], aggs[2], aggs[2], h, w2, b2, w,
      b, w1n)


def _tc_pool(batch3, h, o1w, o1b, o2w, o2b):
    return pl.pallas_call(
        _pool_kernel,
        grid=(GRID_N,),
        in_specs=[
            pl.BlockSpec((1, 1, BN), lambda i: (i, 0, 0)),
            pl.BlockSpec((BN, H), lambda i: (i, 0)),
            _full_spec((H, H)),
            _full_spec((1, H)),
            _full_spec((H, H)),
            _full_spec((1, H)),
        ],
        out_specs=pl.BlockSpec((NG, H), lambda i: (0, 0)),
        out_shape=jax.ShapeDtypeStruct((NG, H), f32),
        scratch_shapes=[pltpu.VMEM((NG, H), f32)],
        compiler_params=pltpu.CompilerParams(
            dimension_semantics=("arbitrary",)),
    )(batch3, h, o1w, o1b, o2w, o2b)


def kernel(z, pos, edge_index, batch, emb, mlp_w1, mlp_b1, mlp_w2, mlp_b2,
           lin1_w, lin2_w, lin2_b, lin_w, lin_b, out1_w, out1_b, out2_w,
           out2_b):
    row = edge_index[0].astype(i32)
    col = edge_index[1].astype(i32)
    px = pos[:, 0]
    py = pos[:, 1]
    pz = pos[:, 2]
    z3 = z.astype(i32).reshape(GRID_N, 1, BN)
    batch3 = batch.astype(i32).reshape(GRID_N, 1, BN)

    embp = jnp.pad(emb, ((0, VP - emb.shape[0]), (0, 0)))
    w1p = jnp.pad(mlp_w1, ((0, 0), (0, GP - G), (0, 0))).astype(jnp.bfloat16)
    offs = jnp.linspace(0.0, CUT, G, dtype=f32)
    offsp = jnp.pad(offs, (0, GP - G), constant_values=1e6).reshape(1, GP)
    o1wp = jnp.pad(out1_w, ((0, 0), (0, H - out1_w.shape[1])))
    o1bp = jnp.pad(out1_b, (0, H - out1_b.shape[0])).reshape(1, H)
    o2wp = jnp.pad(out2_w, ((0, H - out2_w.shape[0]), (0, H - out2_w.shape[1])))
    o2bp = jnp.pad(out2_b, (0, H - out2_b.shape[0])).reshape(1, H)
    zeros_nh = jnp.zeros((N, H), f32)
    _geom, _gather_rows, _scatter_add = _sc_kernels()

    d2 = _geom(px, py, pz, row, col)
    d2g = d2.reshape(GRID_E, 1, BE)

    h, x = _tc_embed(z3, embp, lin1_w[0])

    eas = [_tc_ea(d2g, offsp, k) for k in range(NCHUNK)]

    for l in range(L):
        b1l = mlp_b1[l].reshape(1, H)
        b2l = mlp_b2[l].reshape(1, H)
        xgs = [None] * NCHUNK
        msgs = [None] * NCHUNK
        aggs = [None] * NCHUNK
        xgs[0] = _gather_rows[0](x, col)
        for k in range(NCHUNK):
            msgs[k] = _tc_filter(eas[k][0], eas[k][1], xgs[k], w1p[l], b1l,
                                 mlp_w2[l], b2l, k)
            if k + 1 < NCHUNK:
                xgs[k + 1] = _gather_rows[k + 1](x, col)
            if k > 0:
                aggs[k - 1] = _scatter_add[k - 1](msgs[k - 1], row, zeros_nh)
        aggs[NCHUNK - 1] = _scatter_add[NCHUNK - 1](msgs[NCHUNK - 1], row,
                                                    zeros_nh)
        w1n = lin1_w[(l + 1) % L]
        h, x = _tc_node([a.reshape(NC, N, H) for a in aggs], h,
                        lin2_w[l], lin2_b[l].reshape(1, H), lin_w[l],
                        lin_b[l].reshape(1, H), w1n)

    outp = _tc_pool(batch3, h, o1wp, o1bp, o2wp, o2bp)
    return outp[:, :1]
